# baseline (speedup 1.0000x reference)
import functools
import math

import jax
import jax.numpy as jnp
from jax import lax
from jax.experimental import pallas as pl
from jax.experimental.pallas import tpu as pltpu
from jax.experimental.pallas import tpu_sc as plsc

E = 8
K = 2
N = 2048
D = 768
H = 3072

TILE = 128
NSPLIT = 8
P = 5120
N_TILES = P // TILE
NC, NS = 2, 16
NW = NC * NS
TOK_W = N // NW

_SQRT2 = math.sqrt(2.0)


def _gelu_exact(v):
    return 0.5 * v * (1.0 + lax.erf(v / _SQRT2))


def _gating_body(x_ref, wg_ref, sp1_ref, sp2_ref, g1_ref, g2_ref,
                 te_ref, tv_ref, i1_s, i2_s, r1_s, r2_s):
    logits = jnp.dot(x_ref[...], wg_ref[...], preferred_element_type=jnp.float32)
    iota = lax.broadcasted_iota(jnp.int32, logits.shape, 1)
    m1 = jnp.max(logits, axis=1, keepdims=True)
    i1 = jnp.min(jnp.where(logits == m1, iota, E), axis=1, keepdims=True)
    masked = jnp.where(iota == i1, -jnp.inf, logits)
    m2 = jnp.max(masked, axis=1, keepdims=True)
    i2 = jnp.min(jnp.where(masked == m2, iota, E), axis=1, keepdims=True)
    a = jnp.exp(m2 - m1)
    lanes16 = jnp.ones((1, 16), jnp.float32)
    g1_ref[...] = (1.0 / (1.0 + a)) * lanes16
    g2_ref[...] = (a / (1.0 + a)) * lanes16
    i1_s[...] = i1
    i2_s[...] = i2

    ri = lax.broadcasted_iota(jnp.int32, (TILE, TILE), 0)
    ci = lax.broadcasted_iota(jnp.int32, (TILE, TILE), 1)
    l_strict = (ri > ci).astype(jnp.float32)
    e_blk = lax.broadcasted_iota(jnp.int32, (TILE, E), 1)

    def block(b, carry):
        oh1 = (i1_s[pl.ds(b * TILE, TILE), :] == e_blk).astype(jnp.float32)
        oh2 = (i2_s[pl.ds(b * TILE, TILE), :] == e_blk).astype(jnp.float32)
        oh = jnp.concatenate([oh1, oh2], axis=1)
        excl = jnp.dot(l_strict, oh, preferred_element_type=jnp.float32) + carry
        r1_s[pl.ds(b * TILE, TILE), :] = (
            jnp.sum(excl[:, :E] * oh1, axis=1, keepdims=True).astype(jnp.int32))
        r2_s[pl.ds(b * TILE, TILE), :] = (
            jnp.sum(excl[:, E:] * oh2, axis=1, keepdims=True).astype(jnp.int32))
        return carry + jnp.sum(oh, axis=0, keepdims=True)

    carry = jnp.zeros((1, 2 * E), jnp.float32)
    for b in range(N // TILE):
        carry = block(b, carry)
    counts1 = carry[:, :E]
    counts = counts1 + carry[:, E:]

    padded = jnp.floor((counts + (TILE - 1)) * (1.0 / TILE)) * TILE
    ri8 = lax.broadcasted_iota(jnp.int32, (E, E), 0)
    ci8 = lax.broadcasted_iota(jnp.int32, (E, E), 1)
    u_incl = (ri8 <= ci8).astype(jnp.float32)
    ends = jnp.dot(padded, u_incl, preferred_element_type=jnp.float32)
    offs = ends - padded

    e_full = lax.broadcasted_iota(jnp.int32, (N, E), 1)
    oh1f = (i1_s[...] == e_full).astype(jnp.float32)
    oh2f = (i2_s[...] == e_full).astype(jnp.float32)
    sp1_ref[...] = (
        jnp.sum(oh1f * offs, axis=1, keepdims=True).astype(jnp.int32)
        + r1_s[...])
    sp2_ref[...] = (
        jnp.sum(oh2f * (offs + counts1), axis=1, keepdims=True).astype(jnp.int32)
        + r2_s[...])

    starts = (lax.broadcasted_iota(jnp.int32, (1, N_TILES), 1)
              * TILE).astype(jnp.float32)
    te = jnp.zeros((1, N_TILES), jnp.int32)
    for e in range(E - 1):
        te = te + (ends[:, e:e + 1] <= starts).astype(jnp.int32)
    te_ref[...] = te
    tv_ref[...] = (starts < ends[:, E - 1:E]).astype(jnp.int32)


def _gating(x, w_gate):
    sds = jax.ShapeDtypeStruct
    return pl.pallas_call(
        _gating_body,
        out_shape=(
            sds((N, 1), jnp.int32), sds((N, 1), jnp.int32),
            sds((N, 16), jnp.float32), sds((N, 16), jnp.float32),
            sds((1, N_TILES), jnp.int32), sds((1, N_TILES), jnp.int32),
        ),
        scratch_shapes=[
            pltpu.VMEM((N, 1), jnp.int32), pltpu.VMEM((N, 1), jnp.int32),
            pltpu.VMEM((N, 1), jnp.int32), pltpu.VMEM((N, 1), jnp.int32),
        ],
    )(x, w_gate)


def _dispatch_body(x_hbm, sp1_hbm, sp2_hbm, xs_hbm, xbuf, idx_v, sems):
    wid = lax.axis_index("s") * NC + lax.axis_index("c")
    pltpu.sync_copy(sp1_hbm.at[wid], idx_v.at[0])
    pltpu.sync_copy(sp2_hbm.at[wid], idx_v.at[1])

    CH = TOK_W // 2
    l0 = pltpu.make_async_copy(x_hbm.at[pl.ds(wid * TOK_W, CH)],
                               xbuf.at[pl.ds(0, CH)], sems.at[0])
    l1 = pltpu.make_async_copy(x_hbm.at[pl.ds(wid * TOK_W + CH, CH)],
                               xbuf.at[pl.ds(CH, CH)], sems.at[1])
    l0.start()
    l1.start()
    l0.wait()
    s10 = pltpu.make_async_copy(xbuf.at[pl.ds(0, CH)],
                                xs_hbm.at[idx_v.at[0, 0]], sems.at[2])
    s20 = pltpu.make_async_copy(xbuf.at[pl.ds(0, CH)],
                                xs_hbm.at[idx_v.at[1, 0]], sems.at[3])
    s10.start()
    s20.start()
    l1.wait()
    s11 = pltpu.make_async_copy(xbuf.at[pl.ds(CH, CH)],
                                xs_hbm.at[idx_v.at[0, 1]], sems.at[0])
    s21 = pltpu.make_async_copy(xbuf.at[pl.ds(CH, CH)],
                                xs_hbm.at[idx_v.at[1, 1]], sems.at[1])
    s11.start()
    s21.start()
    s10.wait()
    s20.wait()
    s11.wait()
    s21.wait()


def _dispatch(x, sp1c, sp2c):
    mesh = plsc.VectorSubcoreMesh(core_axis_name="c", subcore_axis_name="s",
                                  num_cores=NC, num_subcores=NS)
    return pl.kernel(
        _dispatch_body,
        out_type=jax.ShapeDtypeStruct((P, D), jnp.float32),
        mesh=mesh,
        scratch_types=[
            pltpu.VMEM((TOK_W, D), jnp.float32),
            pltpu.VMEM((2, 2, TOK_W // 2), jnp.int32),
            pltpu.SemaphoreType.DMA((4,)),
        ],
    )(x, sp1c, sp2c)


def _mlp_body(te_ref, tv_ref, par_ref, chg_ref, nx_ref, hn_ref,
              xs_ref, W1_hbm, b1_ref, W2_hbm, b2_ref, out_ref,
              w1b, w2b, s1, s2):
    i = pl.program_id(0)
    e = te_ref[i]
    par = par_ref[i]

    def w_copies(ex, slot):
        cs = []
        for c in range(NSPLIT):
            d0, dw = c * (D // NSPLIT), D // NSPLIT
            h0, hw = c * (H // NSPLIT), H // NSPLIT
            cs.append(pltpu.make_async_copy(
                W1_hbm.at[ex, pl.ds(d0, dw)], w1b.at[slot, pl.ds(d0, dw)],
                s1.at[c]))
            cs.append(pltpu.make_async_copy(
                W2_hbm.at[ex, pl.ds(h0, hw)], w2b.at[slot, pl.ds(h0, hw)],
                s2.at[c]))
        return cs

    @pl.when(i == 0)
    def _first():
        cs = w_copies(e, 0)
        for c in cs:
            c.start()
        for c in cs:
            c.wait()

    @pl.when((i > 0) & (chg_ref[i] == 1))
    def _arrive():
        for c in w_copies(e, par):
            c.wait()

    @pl.when(((i == 0) | (chg_ref[i] == 1)) & (hn_ref[i] == 1))
    def _launch_next():
        for c in w_copies(nx_ref[i], 1 - par):
            c.start()

    @pl.when(tv_ref[i] != 0)
    def _run():
        h = _gelu_exact(
            jnp.dot(xs_ref[...], w1b[par], preferred_element_type=jnp.float32)
            + b1_ref[0])
        out_ref[...] = (
            jnp.dot(h, w2b[par], preferred_element_type=jnp.float32)
            + b2_ref[0])


def _grouped_mlp(te, tv, par, chg, nx, hn, x_sorted, W1, b1, W2, b2):
    hbm = pl.BlockSpec(memory_space=pltpu.MemorySpace.HBM)
    grid_spec = pltpu.PrefetchScalarGridSpec(
        num_scalar_prefetch=6,
        grid=(N_TILES,),
        in_specs=[
            pl.BlockSpec((TILE, D), lambda i, *s: (i, 0)),
            hbm,
            pl.BlockSpec((1, 1, H), lambda i, te, *s: (te[i], 0, 0)),
            hbm,
            pl.BlockSpec((1, 1, D), lambda i, te, *s: (te[i], 0, 0)),
        ],
        out_specs=pl.BlockSpec((TILE, D), lambda i, *s: (i, 0)),
        scratch_shapes=[
            pltpu.VMEM((2, D, H), jnp.float32),
            pltpu.VMEM((2, H, D), jnp.float32),
            pltpu.SemaphoreType.DMA((NSPLIT,)),
            pltpu.SemaphoreType.DMA((NSPLIT,)),
        ],
    )
    return pl.pallas_call(
        _mlp_body,
        grid_spec=grid_spec,
        out_shape=jax.ShapeDtypeStruct((P, D), jnp.float32),
        compiler_params=pltpu.CompilerParams(
            dimension_semantics=("arbitrary",),
        ),
    )(te, tv, par, chg, nx, hn, x_sorted, W1,
      b1.reshape(E, 1, H), W2, b2.reshape(E, 1, D))


def _combine_body(out_hbm, sp1_hbm, sp2_hbm, g1_hbm, g2_hbm, y_hbm,
                  r1b, r2b, idx_v, g_v, sems):
    wid = lax.axis_index("s") * NC + lax.axis_index("c")
    pltpu.sync_copy(sp1_hbm.at[wid], idx_v.at[0])
    pltpu.sync_copy(sp2_hbm.at[wid], idx_v.at[1])
    pltpu.sync_copy(g1_hbm.at[wid], g_v.at[0])
    pltpu.sync_copy(g2_hbm.at[wid], g_v.at[1])

    CH = TOK_W // 2
    gets = [
        pltpu.make_async_copy(out_hbm.at[idx_v.at[0, pl.ds(0, CH)]],
                              r1b.at[pl.ds(0, CH)], sems.at[0]),
        pltpu.make_async_copy(out_hbm.at[idx_v.at[1, pl.ds(0, CH)]],
                              r2b.at[pl.ds(0, CH)], sems.at[1]),
        pltpu.make_async_copy(out_hbm.at[idx_v.at[0, pl.ds(CH, CH)]],
                              r1b.at[pl.ds(CH, CH)], sems.at[2]),
        pltpu.make_async_copy(out_hbm.at[idx_v.at[1, pl.ds(CH, CH)]],
                              r2b.at[pl.ds(CH, CH)], sems.at[3]),
    ]
    for g in gets:
        g.start()

    def token(j, _):
        ga = g_v[0, j, :]
        gb = g_v[1, j, :]
        for c in range(D // 16):
            av = r1b[j, pl.ds(c * 16, 16)]
            bv = r2b[j, pl.ds(c * 16, 16)]
            r1b[j, pl.ds(c * 16, 16)] = ga * av + gb * bv
        return 0

    gets[0].wait()
    gets[1].wait()
    lax.fori_loop(0, CH, token, 0)
    put0 = pltpu.make_async_copy(r1b.at[pl.ds(0, CH)],
                                 y_hbm.at[pl.ds(wid * TOK_W, CH)], sems.at[0])
    put0.start()
    gets[2].wait()
    gets[3].wait()
    lax.fori_loop(CH, TOK_W, token, 0)
    put1 = pltpu.make_async_copy(r1b.at[pl.ds(CH, CH)],
                                 y_hbm.at[pl.ds(wid * TOK_W + CH, CH)],
                                 sems.at[1])
    put1.start()
    put0.wait()
    put1.wait()


def _combine(out_all, sp1, sp2, g1, g2):
    mesh = plsc.VectorSubcoreMesh(core_axis_name="c", subcore_axis_name="s",
                                  num_cores=NC, num_subcores=NS)
    return pl.kernel(
        _combine_body,
        out_type=jax.ShapeDtypeStruct((N, D), jnp.float32),
        mesh=mesh,
        scratch_types=[
            pltpu.VMEM((TOK_W, D), jnp.float32),
            pltpu.VMEM((TOK_W, D), jnp.float32),
            pltpu.VMEM((2, TOK_W), jnp.int32),
            pltpu.VMEM((2, TOK_W, 16), jnp.float32),
            pltpu.SemaphoreType.DMA((4,)),
        ],
    )(out_all, sp1, sp2, g1, g2)


@jax.jit
def kernel(x, w_gate, W1, b1, W2, b2):
    sp1, sp2, g1, g2, te, tv = _gating(x, w_gate)
    sp1 = sp1.reshape(NW, TOK_W)
    sp2 = sp2.reshape(NW, TOK_W)
    g1 = g1.reshape(NW, TOK_W, 16)
    g2 = g2.reshape(NW, TOK_W, 16)

    te_a, tv_a = te[0], tv[0]
    chg = jnp.concatenate(
        [jnp.zeros((1,), jnp.int32), (te_a[1:] != te_a[:-1]).astype(jnp.int32)])
    par = (jnp.cumsum(chg) % 2).astype(jnp.int32)
    idx = jnp.arange(N_TILES, dtype=jnp.int32)
    big = jnp.where(chg == 1, idx, N_TILES + 1)
    sufmin = lax.associative_scan(jnp.minimum, big, reverse=True)
    nxtb = jnp.concatenate([sufmin[1:], jnp.full((1,), N_TILES + 1, jnp.int32)])
    hn = (nxtb <= N_TILES).astype(jnp.int32)
    nx = te_a[jnp.clip(nxtb, 0, N_TILES - 1)]

    x_sorted = _dispatch(x, sp1.reshape(NW, 2, TOK_W // 2),
                         sp2.reshape(NW, 2, TOK_W // 2))
    out_all = _grouped_mlp(te_a, tv_a, par, chg, nx, hn,
                           x_sorted, W1, b1, W2, b2)
    return _combine(out_all, sp1, sp2, g1, g2)


# speedup vs baseline: 1.0102x; 1.0102x over previous
import functools
import math

import jax
import jax.numpy as jnp
from jax import lax
from jax.experimental import pallas as pl
from jax.experimental.pallas import tpu as pltpu
from jax.experimental.pallas import tpu_sc as plsc

E = 8
K = 2
N = 2048
D = 768
H = 3072

TILE = 128
NSPLIT = 8
P = 5120
N_TILES = P // TILE
NC, NS = 2, 16
NW = NC * NS
TOK_W = N // NW

_SQRT2 = math.sqrt(2.0)


def _gelu_exact(v):
    return 0.5 * v * (1.0 + lax.erf(v / _SQRT2))


def _gating_body(x_ref, wg_ref, sp1_ref, sp2_ref, g1_ref, g2_ref,
                 te_ref, tv_ref, i1_s, i2_s, r1_s, r2_s):
    logits = jnp.dot(x_ref[...], wg_ref[...], preferred_element_type=jnp.float32)
    iota = lax.broadcasted_iota(jnp.int32, logits.shape, 1)
    m1 = jnp.max(logits, axis=1, keepdims=True)
    i1 = jnp.min(jnp.where(logits == m1, iota, E), axis=1, keepdims=True)
    masked = jnp.where(iota == i1, -jnp.inf, logits)
    m2 = jnp.max(masked, axis=1, keepdims=True)
    i2 = jnp.min(jnp.where(masked == m2, iota, E), axis=1, keepdims=True)
    a = jnp.exp(m2 - m1)
    lanes16 = jnp.ones((1, 16), jnp.float32)
    g1_ref[...] = (1.0 / (1.0 + a)) * lanes16
    g2_ref[...] = (a / (1.0 + a)) * lanes16
    i1_s[...] = i1
    i2_s[...] = i2

    ri = lax.broadcasted_iota(jnp.int32, (TILE, TILE), 0)
    ci = lax.broadcasted_iota(jnp.int32, (TILE, TILE), 1)
    l_strict = (ri > ci).astype(jnp.float32)
    e_blk = lax.broadcasted_iota(jnp.int32, (TILE, E), 1)

    def block(b, carry):
        oh1 = (i1_s[pl.ds(b * TILE, TILE), :] == e_blk).astype(jnp.float32)
        oh2 = (i2_s[pl.ds(b * TILE, TILE), :] == e_blk).astype(jnp.float32)
        oh = jnp.concatenate([oh1, oh2], axis=1)
        excl = jnp.dot(l_strict, oh, preferred_element_type=jnp.float32) + carry
        r1_s[pl.ds(b * TILE, TILE), :] = (
            jnp.sum(excl[:, :E] * oh1, axis=1, keepdims=True).astype(jnp.int32))
        r2_s[pl.ds(b * TILE, TILE), :] = (
            jnp.sum(excl[:, E:] * oh2, axis=1, keepdims=True).astype(jnp.int32))
        return carry + jnp.sum(oh, axis=0, keepdims=True)

    carry = jnp.zeros((1, 2 * E), jnp.float32)
    for b in range(N // TILE):
        carry = block(b, carry)
    counts1 = carry[:, :E]
    counts = counts1 + carry[:, E:]

    padded = jnp.floor((counts + (TILE - 1)) * (1.0 / TILE)) * TILE
    ri8 = lax.broadcasted_iota(jnp.int32, (E, E), 0)
    ci8 = lax.broadcasted_iota(jnp.int32, (E, E), 1)
    u_incl = (ri8 <= ci8).astype(jnp.float32)
    ends = jnp.dot(padded, u_incl, preferred_element_type=jnp.float32)
    offs = ends - padded

    e_full = lax.broadcasted_iota(jnp.int32, (N, E), 1)
    oh1f = (i1_s[...] == e_full).astype(jnp.float32)
    oh2f = (i2_s[...] == e_full).astype(jnp.float32)
    sp1_ref[...] = (
        jnp.sum(oh1f * offs, axis=1, keepdims=True).astype(jnp.int32)
        + r1_s[...])
    sp2_ref[...] = (
        jnp.sum(oh2f * (offs + counts1), axis=1, keepdims=True).astype(jnp.int32)
        + r2_s[...])

    starts = (lax.broadcasted_iota(jnp.int32, (1, N_TILES), 1)
              * TILE).astype(jnp.float32)
    te = jnp.zeros((1, N_TILES), jnp.int32)
    for e in range(E - 1):
        te = te + (ends[:, e:e + 1] <= starts).astype(jnp.int32)
    te_ref[...] = te
    tv_ref[...] = (starts < ends[:, E - 1:E]).astype(jnp.int32)


def _gating(x, w_gate):
    sds = jax.ShapeDtypeStruct
    return pl.pallas_call(
        _gating_body,
        out_shape=(
            sds((N, 1), jnp.int32), sds((N, 1), jnp.int32),
            sds((N, 16), jnp.float32), sds((N, 16), jnp.float32),
            sds((1, N_TILES), jnp.int32), sds((1, N_TILES), jnp.int32),
        ),
        scratch_shapes=[
            pltpu.VMEM((N, 1), jnp.int32), pltpu.VMEM((N, 1), jnp.int32),
            pltpu.VMEM((N, 1), jnp.int32), pltpu.VMEM((N, 1), jnp.int32),
        ],
    )(x, w_gate)


def _dispatch_body(x_hbm, sp1_hbm, sp2_hbm, xs_hbm, xbuf, idx_v):
    wid = lax.axis_index("s") * NC + lax.axis_index("c")
    pltpu.sync_copy(sp1_hbm.at[wid], idx_v.at[0])
    pltpu.sync_copy(sp2_hbm.at[wid], idx_v.at[1])
    pltpu.sync_copy(x_hbm.at[pl.ds(wid * TOK_W, TOK_W)], xbuf)
    pltpu.sync_copy(xbuf, xs_hbm.at[idx_v.at[0]])
    pltpu.sync_copy(xbuf, xs_hbm.at[idx_v.at[1]])


def _dispatch(x, sp1, sp2):
    mesh = plsc.VectorSubcoreMesh(core_axis_name="c", subcore_axis_name="s",
                                  num_cores=NC, num_subcores=NS)
    return pl.kernel(
        _dispatch_body,
        out_type=jax.ShapeDtypeStruct((P, D), jnp.float32),
        mesh=mesh,
        scratch_types=[
            pltpu.VMEM((TOK_W, D), jnp.float32),
            pltpu.VMEM((2, TOK_W), jnp.int32),
        ],
    )(x, sp1, sp2)


def _mlp_body(te_ref, tv_ref, par_ref, chg_ref, nx_ref, hn_ref,
              xs_ref, W1_hbm, b1_ref, W2_hbm, b2_ref, out_ref,
              w1b, w2b, s1, s2):
    i = pl.program_id(0)
    e = te_ref[i]
    par = par_ref[i]

    def w_copies(ex, slot):
        cs = []
        for c in range(NSPLIT):
            d0, dw = c * (D // NSPLIT), D // NSPLIT
            h0, hw = c * (H // NSPLIT), H // NSPLIT
            cs.append(pltpu.make_async_copy(
                W1_hbm.at[ex, pl.ds(d0, dw)], w1b.at[slot, pl.ds(d0, dw)],
                s1.at[c]))
            cs.append(pltpu.make_async_copy(
                W2_hbm.at[ex, pl.ds(h0, hw)], w2b.at[slot, pl.ds(h0, hw)],
                s2.at[c]))
        return cs

    @pl.when(i == 0)
    def _first():
        cs = w_copies(e, 0)
        for c in cs:
            c.start()
        for c in cs:
            c.wait()

    @pl.when((i > 0) & (chg_ref[i] == 1))
    def _arrive():
        for c in w_copies(e, par):
            c.wait()

    @pl.when(((i == 0) | (chg_ref[i] == 1)) & (hn_ref[i] == 1))
    def _launch_next():
        for c in w_copies(nx_ref[i], 1 - par):
            c.start()

    @pl.when(tv_ref[i] != 0)
    def _run():
        h = _gelu_exact(
            jnp.dot(xs_ref[...], w1b[par], preferred_element_type=jnp.float32)
            + b1_ref[0])
        out_ref[...] = (
            jnp.dot(h, w2b[par], preferred_element_type=jnp.float32)
            + b2_ref[0])


def _grouped_mlp(te, tv, par, chg, nx, hn, x_sorted, W1, b1, W2, b2):
    hbm = pl.BlockSpec(memory_space=pltpu.MemorySpace.HBM)
    grid_spec = pltpu.PrefetchScalarGridSpec(
        num_scalar_prefetch=6,
        grid=(N_TILES,),
        in_specs=[
            pl.BlockSpec((TILE, D), lambda i, *s: (i, 0)),
            hbm,
            pl.BlockSpec((1, 1, H), lambda i, te, *s: (te[i], 0, 0)),
            hbm,
            pl.BlockSpec((1, 1, D), lambda i, te, *s: (te[i], 0, 0)),
        ],
        out_specs=pl.BlockSpec((TILE, D), lambda i, *s: (i, 0)),
        scratch_shapes=[
            pltpu.VMEM((2, D, H), jnp.float32),
            pltpu.VMEM((2, H, D), jnp.float32),
            pltpu.SemaphoreType.DMA((NSPLIT,)),
            pltpu.SemaphoreType.DMA((NSPLIT,)),
        ],
    )
    return pl.pallas_call(
        _mlp_body,
        grid_spec=grid_spec,
        out_shape=jax.ShapeDtypeStruct((P, D), jnp.float32),
        compiler_params=pltpu.CompilerParams(
            dimension_semantics=("arbitrary",),
        ),
    )(te, tv, par, chg, nx, hn, x_sorted, W1,
      b1.reshape(E, 1, H), W2, b2.reshape(E, 1, D))


def _combine_body(out_hbm, sp1_hbm, sp2_hbm, g1_hbm, g2_hbm, y_hbm,
                  r1b, r2b, idx_v, g_v, sems):
    wid = lax.axis_index("s") * NC + lax.axis_index("c")
    pltpu.sync_copy(sp1_hbm.at[wid], idx_v.at[0])
    pltpu.sync_copy(sp2_hbm.at[wid], idx_v.at[1])
    pltpu.sync_copy(g1_hbm.at[wid], g_v.at[0])
    pltpu.sync_copy(g2_hbm.at[wid], g_v.at[1])

    CH = TOK_W // 2
    gets = [
        pltpu.make_async_copy(out_hbm.at[idx_v.at[0, pl.ds(0, CH)]],
                              r1b.at[pl.ds(0, CH)], sems.at[0]),
        pltpu.make_async_copy(out_hbm.at[idx_v.at[1, pl.ds(0, CH)]],
                              r2b.at[pl.ds(0, CH)], sems.at[1]),
        pltpu.make_async_copy(out_hbm.at[idx_v.at[0, pl.ds(CH, CH)]],
                              r1b.at[pl.ds(CH, CH)], sems.at[2]),
        pltpu.make_async_copy(out_hbm.at[idx_v.at[1, pl.ds(CH, CH)]],
                              r2b.at[pl.ds(CH, CH)], sems.at[3]),
    ]
    for g in gets:
        g.start()

    def token(j, _):
        ga = g_v[0, j, :]
        gb = g_v[1, j, :]
        for c in range(D // 16):
            av = r1b[j, pl.ds(c * 16, 16)]
            bv = r2b[j, pl.ds(c * 16, 16)]
            r1b[j, pl.ds(c * 16, 16)] = ga * av + gb * bv
        return 0

    gets[0].wait()
    gets[1].wait()
    lax.fori_loop(0, CH, token, 0)
    put0 = pltpu.make_async_copy(r1b.at[pl.ds(0, CH)],
                                 y_hbm.at[pl.ds(wid * TOK_W, CH)], sems.at[0])
    put0.start()
    gets[2].wait()
    gets[3].wait()
    lax.fori_loop(CH, TOK_W, token, 0)
    put1 = pltpu.make_async_copy(r1b.at[pl.ds(CH, CH)],
                                 y_hbm.at[pl.ds(wid * TOK_W + CH, CH)],
                                 sems.at[1])
    put1.start()
    put0.wait()
    put1.wait()


def _combine(out_all, sp1, sp2, g1, g2):
    mesh = plsc.VectorSubcoreMesh(core_axis_name="c", subcore_axis_name="s",
                                  num_cores=NC, num_subcores=NS)
    return pl.kernel(
        _combine_body,
        out_type=jax.ShapeDtypeStruct((N, D), jnp.float32),
        mesh=mesh,
        scratch_types=[
            pltpu.VMEM((TOK_W, D), jnp.float32),
            pltpu.VMEM((TOK_W, D), jnp.float32),
            pltpu.VMEM((2, TOK_W), jnp.int32),
            pltpu.VMEM((2, TOK_W, 16), jnp.float32),
            pltpu.SemaphoreType.DMA((4,)),
        ],
    )(out_all, sp1, sp2, g1, g2)


@jax.jit
def kernel(x, w_gate, W1, b1, W2, b2):
    sp1, sp2, g1, g2, te, tv = _gating(x, w_gate)
    sp1 = sp1.reshape(NW, TOK_W)
    sp2 = sp2.reshape(NW, TOK_W)
    g1 = g1.reshape(NW, TOK_W, 16)
    g2 = g2.reshape(NW, TOK_W, 16)

    te_a, tv_a = te[0], tv[0]
    chg = jnp.concatenate(
        [jnp.zeros((1,), jnp.int32), (te_a[1:] != te_a[:-1]).astype(jnp.int32)])
    par = (jnp.cumsum(chg) % 2).astype(jnp.int32)
    idx = jnp.arange(N_TILES, dtype=jnp.int32)
    big = jnp.where(chg == 1, idx, N_TILES + 1)
    sufmin = lax.associative_scan(jnp.minimum, big, reverse=True)
    nxtb = jnp.concatenate([sufmin[1:], jnp.full((1,), N_TILES + 1, jnp.int32)])
    hn = (nxtb <= N_TILES).astype(jnp.int32)
    nx = te_a[jnp.clip(nxtb, 0, N_TILES - 1)]

    x_sorted = _dispatch(x, sp1, sp2)
    out_all = _grouped_mlp(te_a, tv_a, par, chg, nx, hn,
                           x_sorted, W1, b1, W2, b2)
    return _combine(out_all, sp1, sp2, g1, g2)
